# baseline (speedup 1.0000x reference)
import jax
import jax.numpy as jnp
from jax.experimental import pallas as pl
from jax.experimental.pallas import tpu as pltpu

_NB = 2048


def _cwr_head_kernel(x_ref, w_ref, b_ref, o_ref):
    o_ref[:] = jax.lax.dot_general(
        x_ref[:], w_ref[:],
        dimension_numbers=(((1,), (1,)), ((), ())),
        preferred_element_type=jnp.float32,
    ) + b_ref[:]


def kernel(x, W, b):
    batch, k = x.shape
    n = W.shape[0]
    return pl.pallas_call(
        _cwr_head_kernel,
        grid=(pl.cdiv(n, _NB),),
        in_specs=[
            pl.BlockSpec((batch, k), lambda i: (0, 0)),
            pl.BlockSpec((_NB, k), lambda i: (i, 0)),
            pl.BlockSpec((1, _NB), lambda i: (0, i)),
        ],
        out_specs=pl.BlockSpec((batch, _NB), lambda i: (0, i)),
        out_shape=jax.ShapeDtypeStruct((batch, n), jnp.float32),
        compiler_params=pltpu.CompilerParams(
            dimension_semantics=("parallel",),
        ),
    )(x, W, b.reshape(1, n))


# speedup vs baseline: 1.0027x; 1.0027x over previous
import jax
import jax.numpy as jnp
from jax.experimental import pallas as pl
from jax.experimental.pallas import tpu as pltpu

_NB = 4096


def _cwr_head_kernel(x_ref, w_ref, b_ref, o_ref):
    o_ref[:] = jax.lax.dot_general(
        x_ref[:], w_ref[:],
        dimension_numbers=(((1,), (1,)), ((), ())),
        preferred_element_type=jnp.float32,
    ) + b_ref[:]


def kernel(x, W, b):
    batch, k = x.shape
    n = W.shape[0]
    return pl.pallas_call(
        _cwr_head_kernel,
        grid=(pl.cdiv(n, _NB),),
        in_specs=[
            pl.BlockSpec((batch, k), lambda i: (0, 0)),
            pl.BlockSpec((_NB, k), lambda i: (i, 0)),
            pl.BlockSpec((1, _NB), lambda i: (0, i)),
        ],
        out_specs=pl.BlockSpec((batch, _NB), lambda i: (0, i)),
        out_shape=jax.ShapeDtypeStruct((batch, n), jnp.float32),
        compiler_params=pltpu.CompilerParams(
            dimension_semantics=("parallel",),
        ),
    )(x, W, b.reshape(1, n))


# speedup vs baseline: 1.0886x; 1.0857x over previous
import jax
import jax.numpy as jnp
from jax.experimental import pallas as pl
from jax.experimental.pallas import tpu as pltpu

_BM = 32


def _cwr_head_kernel(x_ref, wt_ref, b_ref, o_ref):
    o_ref[:] = jax.lax.dot_general(
        x_ref[:], wt_ref[:],
        dimension_numbers=(((1,), (0,)), ((), ())),
        preferred_element_type=jnp.float32,
    ) + b_ref[:]


def kernel(x, W, b):
    batch, k = x.shape
    n = W.shape[0]
    return pl.pallas_call(
        _cwr_head_kernel,
        grid=(batch // _BM,),
        in_specs=[
            pl.BlockSpec((_BM, k), lambda i: (i, 0)),
            pl.BlockSpec((k, n), lambda i: (0, 0)),
            pl.BlockSpec((1, n), lambda i: (0, 0)),
        ],
        out_specs=pl.BlockSpec((_BM, n), lambda i: (i, 0)),
        out_shape=jax.ShapeDtypeStruct((batch, n), jnp.float32),
        compiler_params=pltpu.CompilerParams(
            dimension_semantics=("arbitrary",),
        ),
    )(x, W.T, b.reshape(1, n))


# speedup vs baseline: 1.0920x; 1.0032x over previous
import jax
import jax.numpy as jnp
from jax.experimental import pallas as pl
from jax.experimental.pallas import tpu as pltpu

_BM = 16
_NBUF = 5


def _cwr_head_kernel(x_ref, wt_ref, b_ref, o_ref, scratch, sems):
    i = pl.program_id(0)
    nsteps = pl.num_programs(0)
    slot = jax.lax.rem(i, _NBUF)

    @pl.when(i >= _NBUF)
    def _wait_reuse():
        pltpu.make_async_copy(
            scratch.at[slot],
            o_ref.at[pl.ds((i - _NBUF) * _BM, _BM), :],
            sems.at[slot],
        ).wait()

    scratch[slot] = jax.lax.dot_general(
        x_ref[:], wt_ref[:],
        dimension_numbers=(((1,), (0,)), ((), ())),
        preferred_element_type=jnp.float32,
    ) + b_ref[:]

    pltpu.make_async_copy(
        scratch.at[slot],
        o_ref.at[pl.ds(i * _BM, _BM), :],
        sems.at[slot],
    ).start()

    @pl.when(i == nsteps - 1)
    def _drain():
        for j in range(_NBUF):
            step = i - j
            s = jax.lax.rem(step, _NBUF)
            pltpu.make_async_copy(
                scratch.at[s],
                o_ref.at[pl.ds(step * _BM, _BM), :],
                sems.at[s],
            ).wait()


def kernel(x, W, b):
    batch, k = x.shape
    n = W.shape[0]
    return pl.pallas_call(
        _cwr_head_kernel,
        grid=(batch // _BM,),
        in_specs=[
            pl.BlockSpec((_BM, k), lambda i: (i, 0)),
            pl.BlockSpec((k, n), lambda i: (0, 0)),
            pl.BlockSpec((1, n), lambda i: (0, 0)),
        ],
        out_specs=pl.BlockSpec(memory_space=pl.ANY),
        out_shape=jax.ShapeDtypeStruct((batch, n), jnp.float32),
        scratch_shapes=[
            pltpu.VMEM((_NBUF, _BM, n), jnp.float32),
            pltpu.SemaphoreType.DMA((_NBUF,)),
        ],
        compiler_params=pltpu.CompilerParams(
            dimension_semantics=("arbitrary",),
        ),
    )(x, W.T, b.reshape(1, n))
